# Initial kernel scaffold; baseline (speedup 1.0000x reference)
import jax
import jax.numpy as jnp
from jax.experimental import pallas as pl


def kernel(ids, table):
    raise NotImplementedError("write your pallas kernel here")


# speedup vs baseline: 1.8427x; 1.8427x over previous
import jax
import jax.numpy as jnp
from jax import lax
from jax.experimental import pallas as pl
from jax.experimental.pallas import tpu as pltpu
from jax.experimental.pallas import tpu_sc as plsc

D = 64
LANES = 128

_info = plsc.get_sparse_core_info()
NC, NS = _info.num_cores, _info.num_subcores
NW = NC * NS

B = 16384 * 50
ROWS = B // LANES
ROWS_PER_W = ROWS // NW
KI = 8
N_OUTER = ROWS_PER_W // KI
CHUNK = KI * LANES


def _gather_body(table_hbm, ids_hbm, out_hbm, idx_v, rows_v, sem):
    wid = lax.axis_index("s") * NC + lax.axis_index("c")
    row_base = wid * ROWS_PER_W

    def step(g, carry):
        r0 = row_base + g * KI
        pltpu.sync_copy(ids_hbm.at[pl.ds(r0, KI)], idx_v)
        copies = [
            pltpu.async_copy(table_hbm.at[idx_v.at[j]],
                             rows_v.at[pl.ds(j * LANES, LANES)], sem)
            for j in range(KI)
        ]
        for c in copies:
            c.wait()
        pltpu.sync_copy(rows_v, out_hbm.at[pl.ds(r0 * LANES, CHUNK)])
        return carry

    lax.fori_loop(0, N_OUTER, step, 0)


@jax.jit
def _embed_lookup(table, ids2d):
    mesh = plsc.VectorSubcoreMesh(core_axis_name="c", subcore_axis_name="s")
    k = pl.kernel(
        _gather_body,
        mesh=mesh,
        out_type=jax.ShapeDtypeStruct((B, D), jnp.float32),
        scratch_types=[
            pltpu.VMEM((KI, LANES), jnp.int32),
            pltpu.VMEM((CHUNK, D), jnp.float32),
            pltpu.SemaphoreType.DMA,
        ],
        compiler_params=pltpu.CompilerParams(use_tc_tiling_on_sc=False),
    )
    return k(table, ids2d)


def kernel(ids, table):
    ids2d = ids.reshape(ROWS, LANES)
    out = _embed_lookup(table, ids2d)
    return out.reshape(ids.shape[0], ids.shape[1], D)


# speedup vs baseline: 1.8545x; 1.0064x over previous
import jax
import jax.numpy as jnp
from jax import lax
from jax.experimental import pallas as pl
from jax.experimental.pallas import tpu as pltpu
from jax.experimental.pallas import tpu_sc as plsc

D = 64
LANES = 128

_info = plsc.get_sparse_core_info()
NC, NS = _info.num_cores, _info.num_subcores
NW = NC * NS

B = 16384 * 50
ROWS = B // LANES
ROWS_PER_W = ROWS // NW
KI = 5
N_OUTER = ROWS_PER_W // KI
CHUNK = KI * LANES


def _gather_body(table_hbm, ids_hbm, out_hbm, idx_v, rows_v, gsem, osem):
    wid = lax.axis_index("s") * NC + lax.axis_index("c")
    row_base = wid * ROWS_PER_W

    def load_idx(c, b):
        pltpu.sync_copy(ids_hbm.at[pl.ds(row_base + c * KI, KI)], idx_v.at[b])

    def fire(c, b):
        for j in range(KI):
            pltpu.async_copy(table_hbm.at[idx_v.at[b, j]],
                             rows_v.at[b, pl.ds(j * LANES, LANES)], gsem.at[b])

    def drain_gather(b):
        for j in range(KI):
            pltpu.make_async_copy(table_hbm.at[idx_v.at[b, j]],
                                  rows_v.at[b, pl.ds(j * LANES, LANES)],
                                  gsem.at[b]).wait()

    def out_copy(c, b):
        return pltpu.make_async_copy(
            rows_v.at[b],
            out_hbm.at[pl.ds((row_base + c * KI) * LANES, CHUNK)],
            osem.at[b])

    load_idx(0, 0)
    fire(0, 0)
    load_idx(1, 1)
    fire(1, 1)

    def step(c, carry):
        b = c % 2
        drain_gather(b)
        out_copy(c, b).start()
        out_copy(c, b).wait()

        @pl.when(c + 2 < N_OUTER)
        def _():
            load_idx(c + 2, b)
            fire(c + 2, b)

        return carry

    lax.fori_loop(0, N_OUTER, step, 0)


@jax.jit
def _embed_lookup(table, ids2d):
    mesh = plsc.VectorSubcoreMesh(core_axis_name="c", subcore_axis_name="s")
    k = pl.kernel(
        _gather_body,
        mesh=mesh,
        out_type=jax.ShapeDtypeStruct((B, D), jnp.float32),
        scratch_types=[
            pltpu.VMEM((2, KI, LANES), jnp.int32),
            pltpu.VMEM((2, CHUNK, D), jnp.float32),
            pltpu.SemaphoreType.DMA((2,)),
            pltpu.SemaphoreType.DMA((2,)),
        ],
        compiler_params=pltpu.CompilerParams(use_tc_tiling_on_sc=False),
    )
    return k(table, ids2d)


def kernel(ids, table):
    ids2d = ids.reshape(ROWS, LANES)
    out = _embed_lookup(table, ids2d)
    return out.reshape(ids.shape[0], ids.shape[1], D)
